# Initial kernel scaffold; baseline (speedup 1.0000x reference)
import jax
import jax.numpy as jnp
from jax.experimental import pallas as pl


def kernel(x, edge_index, batch, W1, b1, W2, b2):
    raise NotImplementedError("write your pallas kernel here")


# speedup vs baseline: 28.8556x; 28.8556x over previous
import functools

import jax
import jax.numpy as jnp
from jax import lax
from jax.experimental import pallas as pl
from jax.experimental.pallas import tpu as pltpu
from jax.experimental.pallas import tpu_sc as plsc

NC = 2
NS = 16
NW = NC * NS
K = 128
BN = 512


def _sc_mesh():
    return plsc.VectorSubcoreMesh(
        core_axis_name="c", subcore_axis_name="s", num_cores=NC, num_subcores=NS
    )


def _make_deg_kernel(n_pad, ch):
    rows = n_pad // NS

    @functools.partial(
        pl.kernel,
        out_type=jax.ShapeDtypeStruct((NC, n_pad), jnp.float32),
        mesh=_sc_mesh(),
        scratch_types=[
            pltpu.VMEM((ch, K), jnp.int32),
            pltpu.VMEM((K,), jnp.float32),
            pltpu.VMEM_SHARED((n_pad,), jnp.float32),
        ],
    )
    def deg_kernel(dst_hbm, zeros_hbm, deg_out, idx_v, ones_v, acc_sh):
        c = lax.axis_index("c")
        s = lax.axis_index("s")
        w = c * NS + s
        pltpu.sync_copy(zeros_hbm.at[pl.ds(s * rows, rows)],
                        acc_sh.at[pl.ds(s * rows, rows)])
        for i in range(K // 16):
            ones_v[pl.ds(i * 16, 16)] = jnp.ones((16,), jnp.float32)
        pltpu.sync_copy(dst_hbm.at[w], idx_v)
        plsc.subcore_barrier()

        def body(j, carry):
            pltpu.sync_copy(ones_v, acc_sh.at[idx_v.at[j]], add=True)
            return carry

        lax.fori_loop(0, ch, body, 0)
        plsc.subcore_barrier()
        pltpu.sync_copy(acc_sh.at[pl.ds(s * rows, rows)],
                        deg_out.at[c, pl.ds(s * rows, rows)])

    return deg_kernel


def _make_edge_kernel(n_pad, ch, d_hid):
    rows = n_pad // NS

    @functools.partial(
        pl.kernel,
        out_type=(
            jax.ShapeDtypeStruct((NC, n_pad, d_hid), jnp.float32),
            jax.ShapeDtypeStruct((NC, n_pad), jnp.float32),
        ),
        mesh=_sc_mesh(),
        scratch_types=[
            pltpu.VMEM((ch, K), jnp.int32),
            pltpu.VMEM((ch, K), jnp.int32),
            pltpu.VMEM((K, d_hid), jnp.float32),
            pltpu.VMEM((K,), jnp.float32),
            pltpu.VMEM_SHARED((n_pad, d_hid), jnp.float32),
            pltpu.VMEM_SHARED((n_pad,), jnp.float32),
        ],
    )
    def edge_kernel(src_hbm, dst_hbm, y_hbm, dinv_hbm, zbig_hbm, zsmall_hbm,
                    a_out, s_out, srcv, dstv, rows_v, dval_v, a_sh, s_sh):
        c = lax.axis_index("c")
        s = lax.axis_index("s")
        w = c * NS + s
        pltpu.sync_copy(zbig_hbm.at[pl.ds(s * rows, rows)],
                        a_sh.at[pl.ds(s * rows, rows)])
        pltpu.sync_copy(zsmall_hbm.at[pl.ds(s * rows, rows)],
                        s_sh.at[pl.ds(s * rows, rows)])
        pltpu.sync_copy(src_hbm.at[w], srcv)
        pltpu.sync_copy(dst_hbm.at[w], dstv)
        plsc.subcore_barrier()

        def body(j, carry):
            pltpu.sync_copy(y_hbm.at[srcv.at[j]], rows_v)
            pltpu.sync_copy(rows_v, a_sh.at[dstv.at[j]], add=True)
            pltpu.sync_copy(dinv_hbm.at[dstv.at[j]], dval_v)
            pltpu.sync_copy(dval_v, s_sh.at[srcv.at[j]], add=True)
            return carry

        lax.fori_loop(0, ch, body, 0)
        plsc.subcore_barrier()
        pltpu.sync_copy(a_sh.at[pl.ds(s * rows, rows)],
                        a_out.at[c, pl.ds(s * rows, rows)])
        pltpu.sync_copy(s_sh.at[pl.ds(s * rows, rows)],
                        s_out.at[c, pl.ds(s * rows, rows)])

    return edge_kernel


def _tc1(x_pad, w1, deg3):
    n_pad, d_in = x_pad.shape
    d_hid = w1.shape[1]

    def body(x_ref, w1_ref, deg_ref, y_ref, dinv_ref):
        dinv = lax.rsqrt(deg_ref[0] + deg_ref[1] + 1.0)
        xw = jnp.dot(x_ref[...], w1_ref[...], preferred_element_type=jnp.float32)
        y_ref[...] = dinv * xw
        dinv_ref[...] = dinv

    return pl.pallas_call(
        body,
        grid=(n_pad // BN,),
        in_specs=[
            pl.BlockSpec((BN, d_in), lambda i: (i, 0)),
            pl.BlockSpec((d_in, d_hid), lambda i: (0, 0)),
            pl.BlockSpec((NC, BN, 1), lambda i: (0, i, 0)),
        ],
        out_specs=[
            pl.BlockSpec((BN, d_hid), lambda i: (i, 0)),
            pl.BlockSpec((BN, 1), lambda i: (i, 0)),
        ],
        out_shape=[
            jax.ShapeDtypeStruct((n_pad, d_hid), jnp.float32),
            jax.ShapeDtypeStruct((n_pad, 1), jnp.float32),
        ],
    )(x_pad, w1, deg3)


def _tc2(a2, y, dinv, s3, b1, w2, b2, n_real):
    nc, n_pad, d_hid = a2.shape
    d_out = w2.shape[1]
    ng = n_pad // BN

    def body(a_ref, y_ref, dinv_ref, s_ref, b1_ref, w2_ref, b2_ref,
             out_ref, acc):
        i = pl.program_id(0)
        a = a_ref[0] + a_ref[1]
        dv = dinv_ref[...]
        h1 = jnp.maximum(dv * (a + y_ref[...]) + b1_ref[...], 0.0)
        sv = s_ref[0] + s_ref[1]
        cvec = dv * (sv + dv)
        row = i * BN + lax.broadcasted_iota(jnp.int32, (BN, 1), 0)
        contrib = jnp.where(row < n_real, cvec * h1, 0.0)
        part = jnp.sum(contrib, axis=0, keepdims=True)

        @pl.when(i == 0)
        def _():
            acc[...] = part

        @pl.when(i > 0)
        def _():
            acc[...] = acc[...] + part

        @pl.when(i == ng - 1)
        def _():
            out_ref[...] = (
                jnp.dot(acc[...], w2_ref[...],
                        preferred_element_type=jnp.float32) * (1.0 / n_real)
                + b2_ref[...]
            )

    return pl.pallas_call(
        body,
        grid=(ng,),
        in_specs=[
            pl.BlockSpec((NC, BN, d_hid), lambda i: (0, i, 0)),
            pl.BlockSpec((BN, d_hid), lambda i: (i, 0)),
            pl.BlockSpec((BN, 1), lambda i: (i, 0)),
            pl.BlockSpec((NC, BN, 1), lambda i: (0, i, 0)),
            pl.BlockSpec((1, d_hid), lambda i: (0, 0)),
            pl.BlockSpec((d_hid, d_out), lambda i: (0, 0)),
            pl.BlockSpec((1, d_out), lambda i: (0, 0)),
        ],
        out_specs=pl.BlockSpec((1, d_out), lambda i: (0, 0)),
        out_shape=jax.ShapeDtypeStruct((1, d_out), jnp.float32),
        scratch_shapes=[pltpu.VMEM((1, d_hid), jnp.float32)],
    )(a2, y, dinv, s3, b1, w2, b2)


def kernel(x, edge_index, batch, W1, b1, W2, b2):
    n, d_in = x.shape
    e = edge_index.shape[1]
    d_hid = W1.shape[1]
    d_out = W2.shape[1]

    n_pad = -(-n // BN) * BN
    e_pad = -(-e // (NW * K)) * (NW * K)
    if e_pad > e and n_pad == n:
        n_pad += BN
    ch = e_pad // (NW * K)

    if e_pad > e:
        pad_idx = n + (jnp.arange(e_pad - e, dtype=jnp.int32) % (n_pad - n))
        src = jnp.concatenate([edge_index[0], pad_idx])
        dst = jnp.concatenate([edge_index[1], pad_idx])
    else:
        src, dst = edge_index[0], edge_index[1]
    src3 = src.reshape(NW, ch, K)
    dst3 = dst.reshape(NW, ch, K)

    zeros_small = jnp.zeros((n_pad,), jnp.float32)
    zeros_big = jnp.zeros((n_pad, d_hid), jnp.float32)

    deg2 = _make_deg_kernel(n_pad, ch)(dst3, zeros_small)
    deg3 = deg2.reshape(NC, n_pad, 1)

    x_pad = jnp.pad(x, ((0, n_pad - n), (0, 0)))
    y, dinv = _tc1(x_pad, W1, deg3)

    a2, s2 = _make_edge_kernel(n_pad, ch, d_hid)(
        src3, dst3, y, dinv.reshape(n_pad), zeros_big, zeros_small
    )

    return _tc2(
        a2, y, dinv, s2.reshape(NC, n_pad, 1),
        b1.reshape(1, d_hid), W2, b2.reshape(1, d_out), n
    )


# speedup vs baseline: 31.7350x; 1.0998x over previous
import functools

import jax
import jax.numpy as jnp
from jax import lax
from jax.experimental import pallas as pl
from jax.experimental.pallas import tpu as pltpu
from jax.experimental.pallas import tpu_sc as plsc

NC = 2
NS = 16
NW = NC * NS
K = 128
BN = 512


def _sc_mesh():
    return plsc.VectorSubcoreMesh(
        core_axis_name="c", subcore_axis_name="s", num_cores=NC, num_subcores=NS
    )


def _make_deg_kernel(n_pad, ch):
    rows = n_pad // NS

    @functools.partial(
        pl.kernel,
        out_type=jax.ShapeDtypeStruct((NC, n_pad), jnp.float32),
        mesh=_sc_mesh(),
        scratch_types=[
            pltpu.VMEM((ch, K), jnp.int32),
            pltpu.VMEM((K,), jnp.float32),
            pltpu.VMEM_SHARED((n_pad,), jnp.float32),
        ],
    )
    def deg_kernel(dst_hbm, zeros_hbm, deg_out, idx_v, ones_v, acc_sh):
        c = lax.axis_index("c")
        s = lax.axis_index("s")
        w = c * NS + s
        pltpu.sync_copy(zeros_hbm.at[pl.ds(s * rows, rows)],
                        acc_sh.at[pl.ds(s * rows, rows)])
        for i in range(K // 16):
            ones_v[pl.ds(i * 16, 16)] = jnp.ones((16,), jnp.float32)
        pltpu.sync_copy(dst_hbm.at[w], idx_v)
        plsc.subcore_barrier()

        def body(j, carry):
            pltpu.sync_copy(ones_v, acc_sh.at[idx_v.at[j]], add=True)
            return carry

        lax.fori_loop(0, ch, body, 0)
        plsc.subcore_barrier()
        pltpu.sync_copy(acc_sh.at[pl.ds(s * rows, rows)],
                        deg_out.at[c, pl.ds(s * rows, rows)])

    return deg_kernel


def _make_edge_kernel(n_pad, ch, d_hid):
    rows = n_pad // NS
    npairs = ch // 2
    dh = d_hid // NC

    @functools.partial(
        pl.kernel,
        out_type=(
            jax.ShapeDtypeStruct((NC, n_pad, dh), jnp.float32),
            jax.ShapeDtypeStruct((n_pad,), jnp.float32),
        ),
        mesh=_sc_mesh(),
        scratch_types=[
            pltpu.VMEM((ch, K), jnp.int32),
            pltpu.VMEM((ch, K), jnp.int32),
            pltpu.VMEM((2, K, dh), jnp.float32),
            pltpu.VMEM((2, K), jnp.float32),
            pltpu.VMEM_SHARED((n_pad, dh), jnp.float32),
            pltpu.VMEM_SHARED((n_pad,), jnp.float32),
            pltpu.SemaphoreType.DMA,
            pltpu.SemaphoreType.DMA,
            pltpu.SemaphoreType.DMA,
            pltpu.SemaphoreType.DMA,
        ],
        compiler_params=pltpu.CompilerParams(use_tc_tiling_on_sc=False),
    )
    def edge_kernel(src_hbm, dst_hbm, y_hbm, dinv_hbm, zbig_hbm, zsmall_hbm,
                    a_out, s_out, srcv, dstv, rows_v, dval_v, a_sh, s_sh,
                    gsem0, gsem1, ssem0, ssem1):
        c = lax.axis_index("c")
        s = lax.axis_index("s")
        on0 = c == 0
        pltpu.sync_copy(zbig_hbm.at[pl.ds(s * rows, rows)],
                        a_sh.at[pl.ds(s * rows, rows)])
        pltpu.sync_copy(zsmall_hbm.at[pl.ds(s * rows, rows)],
                        s_sh.at[pl.ds(s * rows, rows)])
        pltpu.sync_copy(src_hbm.at[s], srcv)
        pltpu.sync_copy(dst_hbm.at[s], dstv)
        plsc.subcore_barrier()

        gsem = (gsem0, gsem1)
        ssem = (ssem0, ssem1)
        ytab = y_hbm.at[c]

        def issue_gather(j, b):
            pltpu.async_copy(ytab.at[srcv.at[j]], rows_v.at[b], gsem[b])

            @pl.when(on0)
            def _():
                pltpu.async_copy(dinv_hbm.at[dstv.at[j]], dval_v.at[b],
                                 gsem[b])

        def wait_gather(j, b):
            pltpu.make_async_copy(ytab.at[srcv.at[j]], rows_v.at[b],
                                  gsem[b]).wait()

            @pl.when(on0)
            def _():
                pltpu.make_async_copy(dinv_hbm.at[dstv.at[j]], dval_v.at[b],
                                      gsem[b]).wait()

        def issue_scatter(j, b):
            pltpu.async_copy(rows_v.at[b], a_sh.at[dstv.at[j]], ssem[b],
                             add=True)

            @pl.when(on0)
            def _():
                pltpu.async_copy(dval_v.at[b], s_sh.at[srcv.at[j]], ssem[b],
                                 add=True)

        def wait_scatter(j, b):
            pltpu.make_async_copy(rows_v.at[b], a_sh.at[dstv.at[j]],
                                  ssem[b]).wait()

            @pl.when(on0)
            def _():
                pltpu.make_async_copy(dval_v.at[b], s_sh.at[srcv.at[j]],
                                      ssem[b]).wait()

        issue_gather(0, 0)

        def body(t, carry):
            j0 = 2 * t
            j1 = j0 + 1
            wait_gather(j0, 0)
            issue_scatter(j0, 0)

            @pl.when(t > 0)
            def _():
                wait_scatter(j0 - 1, 1)

            issue_gather(j1, 1)
            wait_gather(j1, 1)
            issue_scatter(j1, 1)

            @pl.when(t < npairs - 1)
            def _():
                wait_scatter(j0, 0)
                issue_gather(j0 + 2, 0)

            return carry

        lax.fori_loop(0, npairs, body, 0)
        wait_scatter(ch - 2, 0)
        wait_scatter(ch - 1, 1)
        plsc.subcore_barrier()
        pltpu.sync_copy(a_sh.at[pl.ds(s * rows, rows)],
                        a_out.at[c, pl.ds(s * rows, rows)])

        @pl.when(on0)
        def _():
            pltpu.sync_copy(s_sh.at[pl.ds(s * rows, rows)],
                            s_out.at[pl.ds(s * rows, rows)])

    return edge_kernel


def _tc1(x_pad, w1, deg3):
    n_pad, d_in = x_pad.shape
    d_hid = w1.shape[1]
    dh = d_hid // NC

    def body(x_ref, w1_ref, deg_ref, y_ref, dinv_ref):
        dinv = lax.rsqrt(deg_ref[0] + deg_ref[1] + 1.0)
        xw = jnp.dot(x_ref[...], w1_ref[...], preferred_element_type=jnp.float32)
        y = dinv * xw
        y_ref[0] = y[:, :dh]
        y_ref[1] = y[:, dh:]
        dinv_ref[...] = dinv

    return pl.pallas_call(
        body,
        grid=(n_pad // BN,),
        in_specs=[
            pl.BlockSpec((BN, d_in), lambda i: (i, 0)),
            pl.BlockSpec((d_in, d_hid), lambda i: (0, 0)),
            pl.BlockSpec((NC, BN, 1), lambda i: (0, i, 0)),
        ],
        out_specs=[
            pl.BlockSpec((NC, BN, dh), lambda i: (0, i, 0)),
            pl.BlockSpec((BN, 1), lambda i: (i, 0)),
        ],
        out_shape=[
            jax.ShapeDtypeStruct((NC, n_pad, dh), jnp.float32),
            jax.ShapeDtypeStruct((n_pad, 1), jnp.float32),
        ],
    )(x_pad, w1, deg3)


def _tc2(a2, y2, dinv, s3, b1, w2, b2, n_real):
    nc, n_pad, dh = a2.shape
    d_hid = nc * dh
    d_out = w2.shape[1]
    ng = n_pad // BN

    def body(a_ref, y_ref, dinv_ref, s_ref, b1_ref, w2_ref, b2_ref,
             out_ref, acc):
        i = pl.program_id(0)
        a = jnp.concatenate([a_ref[0], a_ref[1]], axis=1)
        yb = jnp.concatenate([y_ref[0], y_ref[1]], axis=1)
        dv = dinv_ref[...]
        h1 = jnp.maximum(dv * (a + yb) + b1_ref[...], 0.0)
        sv = s_ref[...]
        cvec = dv * (sv + dv)
        row = i * BN + lax.broadcasted_iota(jnp.int32, (BN, 1), 0)
        contrib = jnp.where(row < n_real, cvec * h1, 0.0)
        part = jnp.sum(contrib, axis=0, keepdims=True)

        @pl.when(i == 0)
        def _():
            acc[...] = part

        @pl.when(i > 0)
        def _():
            acc[...] = acc[...] + part

        @pl.when(i == ng - 1)
        def _():
            out_ref[...] = (
                jnp.dot(acc[...], w2_ref[...],
                        preferred_element_type=jnp.float32) * (1.0 / n_real)
                + b2_ref[...]
            )

    return pl.pallas_call(
        body,
        grid=(ng,),
        in_specs=[
            pl.BlockSpec((NC, BN, dh), lambda i: (0, i, 0)),
            pl.BlockSpec((NC, BN, dh), lambda i: (0, i, 0)),
            pl.BlockSpec((BN, 1), lambda i: (i, 0)),
            pl.BlockSpec((BN, 1), lambda i: (i, 0)),
            pl.BlockSpec((1, d_hid), lambda i: (0, 0)),
            pl.BlockSpec((d_hid, d_out), lambda i: (0, 0)),
            pl.BlockSpec((1, d_out), lambda i: (0, 0)),
        ],
        out_specs=pl.BlockSpec((1, d_out), lambda i: (0, 0)),
        out_shape=jax.ShapeDtypeStruct((1, d_out), jnp.float32),
        scratch_shapes=[pltpu.VMEM((1, d_hid), jnp.float32)],
    )(a2, y2, dinv, s3, b1, w2, b2)


def kernel(x, edge_index, batch, W1, b1, W2, b2):
    n, d_in = x.shape
    e = edge_index.shape[1]
    d_hid = W1.shape[1]
    d_out = W2.shape[1]

    n_pad = -(-n // BN) * BN
    e_pad = -(-e // (NS * K * 2)) * (NS * K * 2)
    if e_pad > e and n_pad == n:
        n_pad += BN
    ch_deg = e_pad // (NW * K)
    ch = e_pad // (NS * K)

    if e_pad > e:
        pad_idx = n + (jnp.arange(e_pad - e, dtype=jnp.int32) % (n_pad - n))
        src = jnp.concatenate([edge_index[0], pad_idx])
        dst = jnp.concatenate([edge_index[1], pad_idx])
    else:
        src, dst = edge_index[0], edge_index[1]

    zeros_small = jnp.zeros((n_pad,), jnp.float32)
    zeros_big = jnp.zeros((n_pad, d_hid // NC), jnp.float32)

    deg2 = _make_deg_kernel(n_pad, ch_deg)(dst.reshape(NW, ch_deg, K),
                                           zeros_small)
    deg3 = deg2.reshape(NC, n_pad, 1)

    x_pad = jnp.pad(x, ((0, n_pad - n), (0, 0)))
    y2, dinv = _tc1(x_pad, W1, deg3)

    a2, s1 = _make_edge_kernel(n_pad, ch, d_hid)(
        src.reshape(NS, ch, K), dst.reshape(NS, ch, K),
        y2, dinv.reshape(n_pad), zeros_big, zeros_small
    )

    return _tc2(
        a2, y2, dinv, s1.reshape(n_pad, 1),
        b1.reshape(1, d_hid), W2, b2.reshape(1, d_out), n
    )


# speedup vs baseline: 39.8413x; 1.2554x over previous
import functools

import jax
import jax.numpy as jnp
from jax import lax
from jax.experimental import pallas as pl
from jax.experimental.pallas import tpu as pltpu
from jax.experimental.pallas import tpu_sc as plsc

NC = 2
NS = 16
NW = NC * NS
K = 128
BN = 512


def _sc_mesh():
    return plsc.VectorSubcoreMesh(
        core_axis_name="c", subcore_axis_name="s", num_cores=NC, num_subcores=NS
    )


def _make_deg_kernel(n_pad, ch):
    rows = n_pad // NS

    @functools.partial(
        pl.kernel,
        out_type=jax.ShapeDtypeStruct((NC, n_pad), jnp.float32),
        mesh=_sc_mesh(),
        scratch_types=[
            pltpu.VMEM((ch, K), jnp.int32),
            pltpu.VMEM((K,), jnp.float32),
            pltpu.VMEM_SHARED((n_pad,), jnp.float32),
        ],
    )
    def deg_kernel(dst_hbm, zeros_hbm, deg_out, idx_v, ones_v, acc_sh):
        c = lax.axis_index("c")
        s = lax.axis_index("s")
        w = c * NS + s
        pltpu.sync_copy(zeros_hbm.at[pl.ds(s * rows, rows)],
                        acc_sh.at[pl.ds(s * rows, rows)])
        for i in range(K // 16):
            ones_v[pl.ds(i * 16, 16)] = jnp.ones((16,), jnp.float32)
        pltpu.sync_copy(dst_hbm.at[w], idx_v)
        plsc.subcore_barrier()

        def body(j, carry):
            pltpu.sync_copy(ones_v, acc_sh.at[idx_v.at[j]], add=True)
            return carry

        lax.fori_loop(0, ch, body, 0)
        plsc.subcore_barrier()
        pltpu.sync_copy(acc_sh.at[pl.ds(s * rows, rows)],
                        deg_out.at[c, pl.ds(s * rows, rows)])

    return deg_kernel


def _make_edge_kernel(n_pad, ch, d_hid):
    rows = n_pad // NS
    ch2 = ch // 2
    npairs = ch2 // 2

    @functools.partial(
        pl.kernel,
        out_type=(
            jax.ShapeDtypeStruct((NC, n_pad, d_hid), jnp.float32),
            jax.ShapeDtypeStruct((NC, n_pad), jnp.float32),
        ),
        mesh=_sc_mesh(),
        scratch_types=[
            pltpu.VMEM((ch2, K), jnp.int32),
            pltpu.VMEM((ch2, K), jnp.int32),
            pltpu.VMEM((2, K, d_hid), jnp.float32),
            pltpu.VMEM((2, K), jnp.float32),
            pltpu.VMEM_SHARED((n_pad, d_hid), jnp.float32),
            pltpu.VMEM_SHARED((n_pad,), jnp.float32),
            pltpu.SemaphoreType.DMA,
            pltpu.SemaphoreType.DMA,
            pltpu.SemaphoreType.DMA,
            pltpu.SemaphoreType.DMA,
        ],
    )
    def edge_kernel(src_hbm, dst_hbm, y_hbm, dinv_hbm, zbig_hbm, zsmall_hbm,
                    a_out, s_out, srcv, dstv, rows_v, dval_v, a_sh, s_sh,
                    gsem0, gsem1, ssem0, ssem1):
        c = lax.axis_index("c")
        s = lax.axis_index("s")
        w = c * NS + s
        pltpu.sync_copy(zbig_hbm.at[pl.ds(s * rows, rows)],
                        a_sh.at[pl.ds(s * rows, rows)])
        pltpu.sync_copy(zsmall_hbm.at[pl.ds(s * rows, rows)],
                        s_sh.at[pl.ds(s * rows, rows)])
        plsc.subcore_barrier()

        gsem = (gsem0, gsem1)
        ssem = (ssem0, ssem1)

        def issue_gather(j, b):
            pltpu.async_copy(y_hbm.at[srcv.at[j]], rows_v.at[b], gsem[b])
            pltpu.async_copy(dinv_hbm.at[dstv.at[j]], dval_v.at[b], gsem[b])

        def wait_gather(j, b):
            pltpu.make_async_copy(y_hbm.at[srcv.at[j]], rows_v.at[b],
                                  gsem[b]).wait()
            pltpu.make_async_copy(dinv_hbm.at[dstv.at[j]], dval_v.at[b],
                                  gsem[b]).wait()

        def issue_scatter(j, b):
            pltpu.async_copy(rows_v.at[b], a_sh.at[dstv.at[j]], ssem[b],
                             add=True)
            pltpu.async_copy(dval_v.at[b], s_sh.at[srcv.at[j]], ssem[b],
                             add=True)

        def wait_scatter(j, b):
            pltpu.make_async_copy(rows_v.at[b], a_sh.at[dstv.at[j]],
                                  ssem[b]).wait()
            pltpu.make_async_copy(dval_v.at[b], s_sh.at[srcv.at[j]],
                                  ssem[b]).wait()

        for p in range(2):
            pltpu.sync_copy(src_hbm.at[w, p], srcv)
            pltpu.sync_copy(dst_hbm.at[w, p], dstv)
            issue_gather(0, 0)

            def body(t, carry):
                j0 = 2 * t
                j1 = j0 + 1
                wait_gather(j0, 0)
                issue_scatter(j0, 0)

                @pl.when(t > 0)
                def _():
                    wait_scatter(j0 - 1, 1)

                issue_gather(j1, 1)
                wait_gather(j1, 1)
                issue_scatter(j1, 1)

                @pl.when(t < npairs - 1)
                def _():
                    wait_scatter(j0, 0)
                    issue_gather(j0 + 2, 0)

                return carry

            lax.fori_loop(0, npairs, body, 0)
            wait_scatter(ch2 - 2, 0)
            wait_scatter(ch2 - 1, 1)

        plsc.subcore_barrier()
        pltpu.sync_copy(a_sh.at[pl.ds(s * rows, rows)],
                        a_out.at[c, pl.ds(s * rows, rows)])
        pltpu.sync_copy(s_sh.at[pl.ds(s * rows, rows)],
                        s_out.at[c, pl.ds(s * rows, rows)])

    return edge_kernel


def _tc1(x_pad, w1, deg3):
    n_pad, d_in = x_pad.shape
    d_hid = w1.shape[1]
    dh = d_hid // NC

    def body(x_ref, w1_ref, deg_ref, y_ref, dinv_ref):
        dinv = lax.rsqrt(deg_ref[0] + deg_ref[1] + 1.0)
        xw = jnp.dot(x_ref[...], w1_ref[...], preferred_element_type=jnp.float32)
        y_ref[...] = dinv * xw
        dinv_ref[...] = dinv

    return pl.pallas_call(
        body,
        grid=(n_pad // BN,),
        in_specs=[
            pl.BlockSpec((BN, d_in), lambda i: (i, 0)),
            pl.BlockSpec((d_in, d_hid), lambda i: (0, 0)),
            pl.BlockSpec((NC, BN, 1), lambda i: (0, i, 0)),
        ],
        out_specs=[
            pl.BlockSpec((BN, d_hid), lambda i: (i, 0)),
            pl.BlockSpec((BN, 1), lambda i: (i, 0)),
        ],
        out_shape=[
            jax.ShapeDtypeStruct((n_pad, d_hid), jnp.float32),
            jax.ShapeDtypeStruct((n_pad, 1), jnp.float32),
        ],
    )(x_pad, w1, deg3)


def _tc2(a2, y, dinv, s3, b1, w2, b2, n_real):
    nc, n_pad, d_hid = a2.shape
    d_out = w2.shape[1]
    ng = n_pad // BN

    def body(a_ref, y_ref, dinv_ref, s_ref, b1_ref, w2_ref, b2_ref,
             out_ref, acc):
        i = pl.program_id(0)
        a = a_ref[0] + a_ref[1]
        dv = dinv_ref[...]
        h1 = jnp.maximum(dv * (a + y_ref[...]) + b1_ref[...], 0.0)
        sv = s_ref[0] + s_ref[1]
        cvec = dv * (sv + dv)
        row = i * BN + lax.broadcasted_iota(jnp.int32, (BN, 1), 0)
        contrib = jnp.where(row < n_real, cvec * h1, 0.0)
        part = jnp.sum(contrib, axis=0, keepdims=True)

        @pl.when(i == 0)
        def _():
            acc[...] = part

        @pl.when(i > 0)
        def _():
            acc[...] = acc[...] + part

        @pl.when(i == ng - 1)
        def _():
            out_ref[...] = (
                jnp.dot(acc[...], w2_ref[...],
                        preferred_element_type=jnp.float32) * (1.0 / n_real)
                + b2_ref[...]
            )

    return pl.pallas_call(
        body,
        grid=(ng,),
        in_specs=[
            pl.BlockSpec((NC, BN, d_hid), lambda i: (0, i, 0)),
            pl.BlockSpec((BN, d_hid), lambda i: (i, 0)),
            pl.BlockSpec((BN, 1), lambda i: (i, 0)),
            pl.BlockSpec((NC, BN, 1), lambda i: (0, i, 0)),
            pl.BlockSpec((1, d_hid), lambda i: (0, 0)),
            pl.BlockSpec((d_hid, d_out), lambda i: (0, 0)),
            pl.BlockSpec((1, d_out), lambda i: (0, 0)),
        ],
        out_specs=pl.BlockSpec((1, d_out), lambda i: (0, 0)),
        out_shape=jax.ShapeDtypeStruct((1, d_out), jnp.float32),
        scratch_shapes=[pltpu.VMEM((1, d_hid), jnp.float32)],
    )(a2, y, dinv, s3, b1, w2, b2)


def kernel(x, edge_index, batch, W1, b1, W2, b2):
    n, d_in = x.shape
    e = edge_index.shape[1]
    d_hid = W1.shape[1]
    d_out = W2.shape[1]

    n_pad = -(-n // BN) * BN
    e_pad = -(-e // (NW * K * 4)) * (NW * K * 4)
    if e_pad > e and n_pad == n:
        n_pad += BN
    ch = e_pad // (NW * K)

    if e_pad > e:
        pad_idx = n + (jnp.arange(e_pad - e, dtype=jnp.int32) % (n_pad - n))
        src = jnp.concatenate([edge_index[0], pad_idx])
        dst = jnp.concatenate([edge_index[1], pad_idx])
    else:
        src, dst = edge_index[0], edge_index[1]

    zeros_small = jnp.zeros((n_pad,), jnp.float32)
    zeros_big = jnp.zeros((n_pad, d_hid), jnp.float32)

    deg2 = _make_deg_kernel(n_pad, ch)(dst.reshape(NW, ch, K), zeros_small)
    deg3 = deg2.reshape(NC, n_pad, 1)

    x_pad = jnp.pad(x, ((0, n_pad - n), (0, 0)))
    y, dinv = _tc1(x_pad, W1, deg3)

    a2, s2 = _make_edge_kernel(n_pad, ch, d_hid)(
        src.reshape(NW, 2, ch // 2, K), dst.reshape(NW, 2, ch // 2, K),
        y, dinv.reshape(n_pad), zeros_big, zeros_small
    )

    return _tc2(
        a2, y, dinv, s2.reshape(NC, n_pad, 1),
        b1.reshape(1, d_hid), W2, b2.reshape(1, d_out), n
    )


# speedup vs baseline: 41.6642x; 1.0458x over previous
import functools

import jax
import jax.numpy as jnp
from jax import lax
from jax.experimental import pallas as pl
from jax.experimental.pallas import tpu as pltpu
from jax.experimental.pallas import tpu_sc as plsc

NC = 2
NS = 16
NW = NC * NS
K = 128
BN = 1024
LN = 128


def _sc_mesh():
    return plsc.VectorSubcoreMesh(
        core_axis_name="c", subcore_axis_name="s", num_cores=NC, num_subcores=NS
    )


def _make_deg_kernel(n_pad, ch):
    rows = n_pad // NS
    ch2 = ch // 2

    @functools.partial(
        pl.kernel,
        out_type=jax.ShapeDtypeStruct((NC, n_pad), jnp.float32),
        mesh=_sc_mesh(),
        scratch_types=[
            pltpu.VMEM((2, ch2, K), jnp.int32),
            pltpu.VMEM((K,), jnp.float32),
            pltpu.VMEM_SHARED((n_pad,), jnp.float32),
        ],
    )
    def deg_kernel(dst_hbm, zeros_hbm, deg_out, idx_v, ones_v, acc_sh):
        c = lax.axis_index("c")
        s = lax.axis_index("s")
        w = c * NS + s
        pltpu.sync_copy(zeros_hbm.at[pl.ds(s * rows, rows)],
                        acc_sh.at[pl.ds(s * rows, rows)])
        for i in range(K // 16):
            ones_v[pl.ds(i * 16, 16)] = jnp.ones((16,), jnp.float32)
        pltpu.sync_copy(dst_hbm.at[w], idx_v)
        plsc.subcore_barrier()

        def body(j, carry):
            pltpu.sync_copy(ones_v, acc_sh.at[idx_v.at[j // ch2, j % ch2]],
                            add=True)
            return carry

        lax.fori_loop(0, ch, body, 0)
        plsc.subcore_barrier()
        pltpu.sync_copy(acc_sh.at[pl.ds(s * rows, rows)],
                        deg_out.at[c, pl.ds(s * rows, rows)])

    return deg_kernel


def _make_edge_kernel(n_pad, ch, d_hid):
    rows = n_pad // NS
    ch2 = ch // 2
    npairs = ch2 // 2

    @functools.partial(
        pl.kernel,
        out_type=(
            jax.ShapeDtypeStruct((NC, n_pad, d_hid), jnp.float32),
            jax.ShapeDtypeStruct((NC, n_pad), jnp.float32),
        ),
        mesh=_sc_mesh(),
        scratch_types=[
            pltpu.VMEM((ch2, K), jnp.int32),
            pltpu.VMEM((ch2, K), jnp.int32),
            pltpu.VMEM((2, K, d_hid), jnp.float32),
            pltpu.VMEM((2, K), jnp.float32),
            pltpu.VMEM_SHARED((n_pad, d_hid), jnp.float32),
            pltpu.VMEM_SHARED((n_pad,), jnp.float32),
            pltpu.SemaphoreType.DMA,
            pltpu.SemaphoreType.DMA,
            pltpu.SemaphoreType.DMA,
            pltpu.SemaphoreType.DMA,
        ],
    )
    def edge_kernel(src_hbm, dst_hbm, y_hbm, dinv_hbm, zbig_hbm, zsmall_hbm,
                    a_out, s_out, srcv, dstv, rows_v, dval_v, a_sh, s_sh,
                    gsem0, gsem1, ssem0, ssem1):
        c = lax.axis_index("c")
        s = lax.axis_index("s")
        w = c * NS + s
        pltpu.sync_copy(zbig_hbm.at[pl.ds(s * rows, rows)],
                        a_sh.at[pl.ds(s * rows, rows)])
        pltpu.sync_copy(zsmall_hbm.at[pl.ds(s * rows, rows)],
                        s_sh.at[pl.ds(s * rows, rows)])
        plsc.subcore_barrier()

        gsem = (gsem0, gsem1)
        ssem = (ssem0, ssem1)

        def issue_gather(j, b):
            pltpu.async_copy(y_hbm.at[srcv.at[j]], rows_v.at[b], gsem[b])
            pltpu.async_copy(dinv_hbm.at[dstv.at[j]], dval_v.at[b], gsem[b])

        def wait_gather(j, b):
            pltpu.make_async_copy(y_hbm.at[srcv.at[j]], rows_v.at[b],
                                  gsem[b]).wait()
            pltpu.make_async_copy(dinv_hbm.at[dstv.at[j]], dval_v.at[b],
                                  gsem[b]).wait()

        def issue_scatter(j, b):
            pltpu.async_copy(rows_v.at[b], a_sh.at[dstv.at[j]], ssem[b],
                             add=True)
            pltpu.async_copy(dval_v.at[b], s_sh.at[srcv.at[j]], ssem[b],
                             add=True)

        def wait_scatter(j, b):
            pltpu.make_async_copy(rows_v.at[b], a_sh.at[dstv.at[j]],
                                  ssem[b]).wait()
            pltpu.make_async_copy(dval_v.at[b], s_sh.at[srcv.at[j]],
                                  ssem[b]).wait()

        for p in range(2):
            pltpu.sync_copy(src_hbm.at[w, p], srcv)
            pltpu.sync_copy(dst_hbm.at[w, p], dstv)
            issue_gather(0, 0)

            def body(t, carry):
                j0 = 2 * t
                j1 = j0 + 1
                wait_gather(j0, 0)
                issue_scatter(j0, 0)

                @pl.when(t > 0)
                def _():
                    wait_scatter(j0 - 1, 1)

                issue_gather(j1, 1)
                wait_gather(j1, 1)
                issue_scatter(j1, 1)

                @pl.when(t < npairs - 1)
                def _():
                    wait_scatter(j0, 0)
                    issue_gather(j0 + 2, 0)

                return carry

            lax.fori_loop(0, npairs, body, 0)
            wait_scatter(ch2 - 2, 0)
            wait_scatter(ch2 - 1, 1)

        plsc.subcore_barrier()
        pltpu.sync_copy(a_sh.at[pl.ds(s * rows, rows)],
                        a_out.at[c, pl.ds(s * rows, rows)])
        pltpu.sync_copy(s_sh.at[pl.ds(s * rows, rows)],
                        s_out.at[c, pl.ds(s * rows, rows)])

    return edge_kernel


def _tc1(x_pad, w1, deg3):
    n_pad, d_in = x_pad.shape
    d_hid = w1.shape[1]

    def body(x_ref, w1_ref, deg_ref, y_ref, dinv_ref):
        dinv = lax.rsqrt(deg_ref[0] + deg_ref[1] + 1.0)
        xw = jnp.dot(x_ref[...], w1_ref[...], preferred_element_type=jnp.float32)
        y_ref[...] = dinv * xw
        dinv_ref[...] = dinv

    return pl.pallas_call(
        body,
        grid=(n_pad // BN,),
        in_specs=[
            pl.BlockSpec((BN, d_in), lambda i: (i, 0)),
            pl.BlockSpec((d_in, d_hid), lambda i: (0, 0)),
            pl.BlockSpec((NC, BN, 1), lambda i: (0, i, 0)),
        ],
        out_specs=[
            pl.BlockSpec((BN, d_hid), lambda i: (i, 0)),
            pl.BlockSpec((BN, 1), lambda i: (i, 0)),
        ],
        out_shape=[
            jax.ShapeDtypeStruct((n_pad, d_hid), jnp.float32),
            jax.ShapeDtypeStruct((n_pad, 1), jnp.float32),
        ],
    )(x_pad, w1, deg3)


def _tc2(a2, y, dinv, s3, b1, w2, b2, n_real):
    nc, n_pad, d_hid = a2.shape
    d_out = w2.shape[1]
    ng = n_pad // BN

    def body(a_ref, y_ref, dinv_ref, s_ref, b1_ref, w2_ref, b2_ref,
             out_ref, acc):
        i = pl.program_id(0)
        a = a_ref[0] + a_ref[1]
        dv = dinv_ref[...]
        h1 = jnp.maximum(dv * (a + y_ref[...]) + b1_ref[...], 0.0)
        sv = s_ref[0] + s_ref[1]
        cvec = dv * (sv + dv)
        row = i * BN + lax.broadcasted_iota(jnp.int32, (BN, 1), 0)
        contrib = jnp.where(row < n_real, cvec * h1, 0.0)
        part = jnp.sum(contrib, axis=0, keepdims=True)

        @pl.when(i == 0)
        def _():
            acc[...] = part

        @pl.when(i > 0)
        def _():
            acc[...] = acc[...] + part

        @pl.when(i == ng - 1)
        def _():
            out_ref[...] = (
                jnp.dot(acc[...], w2_ref[...],
                        preferred_element_type=jnp.float32) * (1.0 / n_real)
                + b2_ref[...]
            )

    return pl.pallas_call(
        body,
        grid=(ng,),
        in_specs=[
            pl.BlockSpec((NC, BN, d_hid), lambda i: (0, i, 0)),
            pl.BlockSpec((BN, d_hid), lambda i: (i, 0)),
            pl.BlockSpec((BN, 1), lambda i: (i, 0)),
            pl.BlockSpec((NC, BN, 1), lambda i: (0, i, 0)),
            pl.BlockSpec((1, d_hid), lambda i: (0, 0)),
            pl.BlockSpec((d_hid, d_out), lambda i: (0, 0)),
            pl.BlockSpec((1, d_out), lambda i: (0, 0)),
        ],
        out_specs=pl.BlockSpec((1, d_out), lambda i: (0, 0)),
        out_shape=jax.ShapeDtypeStruct((1, d_out), jnp.float32),
        scratch_shapes=[pltpu.VMEM((1, d_hid), jnp.float32)],
    )(a2, y, dinv, s3, b1, w2, b2)


def kernel(x, edge_index, batch, W1, b1, W2, b2):
    n, d_in = x.shape
    e = edge_index.shape[1]
    d_hid = W1.shape[1]
    d_out = W2.shape[1]

    n_pad = -(-n // BN) * BN
    e_pad = -(-e // (NW * K * 4)) * (NW * K * 4)
    if e_pad > e and n_pad == n:
        n_pad += BN
    ch = e_pad // (NW * K)

    if e_pad > e:
        pad_idx = n + (jnp.arange(e_pad - e, dtype=jnp.int32) % (n_pad - n))
        src = jnp.concatenate([edge_index[0], pad_idx])
        dst = jnp.concatenate([edge_index[1], pad_idx])
    else:
        src, dst = edge_index[0], edge_index[1]

    zeros_small = jnp.zeros((n_pad,), jnp.float32)
    zeros_big = jnp.zeros((n_pad, d_hid), jnp.float32)

    src4 = src.reshape(NW, 2, ch // 2, K)
    dst4 = dst.reshape(NW, 2, ch // 2, K)

    deg2 = _make_deg_kernel(n_pad, ch)(dst4, zeros_small)

    x_pad = jnp.pad(x, ((0, n_pad - n), (0, 0)))
    y, dinv = _tc1(x_pad, W1, deg2.reshape(NC, n_pad, 1))

    a2, s2 = _make_edge_kernel(n_pad, ch, d_hid)(
        src4, dst4, y, dinv.reshape(n_pad), zeros_big, zeros_small
    )

    return _tc2(
        a2, y, dinv, s2.reshape(NC, n_pad, 1),
        b1.reshape(1, d_hid), W2, b2.reshape(1, d_out), n
    )


# speedup vs baseline: 45.7092x; 1.0971x over previous
import functools

import jax
import jax.numpy as jnp
from jax import lax
from jax.experimental import pallas as pl
from jax.experimental.pallas import tpu as pltpu
from jax.experimental.pallas import tpu_sc as plsc

NC = 2
NS = 16
NW = NC * NS
K = 80
BN = 1024
LN = 128


def _sc_mesh():
    return plsc.VectorSubcoreMesh(
        core_axis_name="c", subcore_axis_name="s", num_cores=NC, num_subcores=NS
    )


def _make_deg_kernel(n_pad, ch):
    rows = n_pad // NS
    ch2 = ch // 2

    @functools.partial(
        pl.kernel,
        out_type=jax.ShapeDtypeStruct((NC, n_pad), jnp.float32),
        mesh=_sc_mesh(),
        scratch_types=[
            pltpu.VMEM((2, ch2, K), jnp.int32),
            pltpu.VMEM((K,), jnp.float32),
            pltpu.VMEM_SHARED((n_pad,), jnp.float32),
        ],
    )
    def deg_kernel(dst_hbm, zeros_hbm, deg_out, idx_v, ones_v, acc_sh):
        c = lax.axis_index("c")
        s = lax.axis_index("s")
        w = c * NS + s
        pltpu.sync_copy(zeros_hbm.at[pl.ds(s * rows, rows)],
                        acc_sh.at[pl.ds(s * rows, rows)])
        for i in range(K // 16):
            ones_v[pl.ds(i * 16, 16)] = jnp.ones((16,), jnp.float32)
        pltpu.sync_copy(dst_hbm.at[w], idx_v)
        plsc.subcore_barrier()

        def body(j, carry):
            pltpu.sync_copy(ones_v, acc_sh.at[idx_v.at[j // ch2, j % ch2]],
                            add=True)
            return carry

        lax.fori_loop(0, ch, body, 0)
        plsc.subcore_barrier()
        pltpu.sync_copy(acc_sh.at[pl.ds(s * rows, rows)],
                        deg_out.at[c, pl.ds(s * rows, rows)])

    return deg_kernel


def _make_edge_kernel(n_pad, ch, d_hid):
    rows = n_pad // NS
    ch2 = ch // 2
    nt = ch2 // 3

    @functools.partial(
        pl.kernel,
        out_type=(
            jax.ShapeDtypeStruct((NC, n_pad, d_hid), jnp.float32),
            jax.ShapeDtypeStruct((NC, n_pad), jnp.float32),
        ),
        mesh=_sc_mesh(),
        scratch_types=[
            pltpu.VMEM((ch2, K), jnp.int32),
            pltpu.VMEM((ch2, K), jnp.int32),
            pltpu.VMEM((3, K, d_hid), jnp.float32),
            pltpu.VMEM((3, K), jnp.float32),
            pltpu.VMEM_SHARED((n_pad, d_hid), jnp.float32),
            pltpu.VMEM_SHARED((n_pad,), jnp.float32),
            pltpu.SemaphoreType.DMA,
            pltpu.SemaphoreType.DMA,
            pltpu.SemaphoreType.DMA,
            pltpu.SemaphoreType.DMA,
            pltpu.SemaphoreType.DMA,
            pltpu.SemaphoreType.DMA,
        ],
    )
    def edge_kernel(src_hbm, dst_hbm, y_hbm, dinv_hbm, zbig_hbm, zsmall_hbm,
                    a_out, s_out, srcv, dstv, rows_v, dval_v, a_sh, s_sh,
                    gsem0, gsem1, gsem2, ssem0, ssem1, ssem2):
        c = lax.axis_index("c")
        s = lax.axis_index("s")
        w = c * NS + s
        pltpu.sync_copy(zbig_hbm.at[pl.ds(s * rows, rows)],
                        a_sh.at[pl.ds(s * rows, rows)])
        pltpu.sync_copy(zsmall_hbm.at[pl.ds(s * rows, rows)],
                        s_sh.at[pl.ds(s * rows, rows)])
        plsc.subcore_barrier()

        gsem = (gsem0, gsem1, gsem2)
        ssem = (ssem0, ssem1, ssem2)

        def issue_gather(j, b):
            pltpu.async_copy(y_hbm.at[srcv.at[j]], rows_v.at[b], gsem[b])
            pltpu.async_copy(dinv_hbm.at[dstv.at[j]], dval_v.at[b], gsem[b])

        def wait_gather(j, b):
            pltpu.make_async_copy(y_hbm.at[srcv.at[j]], rows_v.at[b],
                                  gsem[b]).wait()
            pltpu.make_async_copy(dinv_hbm.at[dstv.at[j]], dval_v.at[b],
                                  gsem[b]).wait()

        def issue_scatter(j, b):
            pltpu.async_copy(rows_v.at[b], a_sh.at[dstv.at[j]], ssem[b],
                             add=True)
            pltpu.async_copy(dval_v.at[b], s_sh.at[srcv.at[j]], ssem[b],
                             add=True)

        def wait_scatter(j, b):
            pltpu.make_async_copy(rows_v.at[b], a_sh.at[dstv.at[j]],
                                  ssem[b]).wait()
            pltpu.make_async_copy(dval_v.at[b], s_sh.at[srcv.at[j]],
                                  ssem[b]).wait()

        for p in range(2):
            pltpu.sync_copy(src_hbm.at[w, p], srcv)
            pltpu.sync_copy(dst_hbm.at[w, p], dstv)
            issue_gather(0, 0)
            issue_gather(1, 1)

            def body(t, carry):
                j0 = 3 * t
                wait_gather(j0, 0)
                issue_scatter(j0, 0)

                @pl.when(t > 0)
                def _():
                    wait_scatter(j0 - 1, 2)

                issue_gather(j0 + 2, 2)
                wait_gather(j0 + 1, 1)
                issue_scatter(j0 + 1, 1)
                wait_scatter(j0, 0)

                @pl.when(t < nt - 1)
                def _():
                    issue_gather(j0 + 3, 0)

                wait_gather(j0 + 2, 2)
                issue_scatter(j0 + 2, 2)
                wait_scatter(j0 + 1, 1)

                @pl.when(t < nt - 1)
                def _():
                    issue_gather(j0 + 4, 1)

                return carry

            lax.fori_loop(0, nt, body, 0)
            wait_scatter(ch2 - 1, 2)

        plsc.subcore_barrier()
        pltpu.sync_copy(a_sh.at[pl.ds(s * rows, rows)],
                        a_out.at[c, pl.ds(s * rows, rows)])
        pltpu.sync_copy(s_sh.at[pl.ds(s * rows, rows)],
                        s_out.at[c, pl.ds(s * rows, rows)])

    return edge_kernel


def _tc1(x_pad, w1, deg3):
    n_pad, d_in = x_pad.shape
    d_hid = w1.shape[1]

    def body(x_ref, w1_ref, deg_ref, y_ref, dinv_ref):
        dinv = lax.rsqrt(deg_ref[0] + deg_ref[1] + 1.0)
        xw = jnp.dot(x_ref[...], w1_ref[...], preferred_element_type=jnp.float32)
        y_ref[...] = dinv * xw
        dinv_ref[...] = dinv

    return pl.pallas_call(
        body,
        grid=(n_pad // BN,),
        in_specs=[
            pl.BlockSpec((BN, d_in), lambda i: (i, 0)),
            pl.BlockSpec((d_in, d_hid), lambda i: (0, 0)),
            pl.BlockSpec((NC, BN, 1), lambda i: (0, i, 0)),
        ],
        out_specs=[
            pl.BlockSpec((BN, d_hid), lambda i: (i, 0)),
            pl.BlockSpec((BN, 1), lambda i: (i, 0)),
        ],
        out_shape=[
            jax.ShapeDtypeStruct((n_pad, d_hid), jnp.float32),
            jax.ShapeDtypeStruct((n_pad, 1), jnp.float32),
        ],
    )(x_pad, w1, deg3)


def _tc2(a2, y, dinv, s3, b1, w2, b2, n_real):
    nc, n_pad, d_hid = a2.shape
    d_out = w2.shape[1]
    ng = n_pad // BN

    def body(a_ref, y_ref, dinv_ref, s_ref, b1_ref, w2_ref, b2_ref,
             out_ref, acc):
        i = pl.program_id(0)
        a = a_ref[0] + a_ref[1]
        dv = dinv_ref[...]
        h1 = jnp.maximum(dv * (a + y_ref[...]) + b1_ref[...], 0.0)
        sv = s_ref[0] + s_ref[1]
        cvec = dv * (sv + dv)
        row = i * BN + lax.broadcasted_iota(jnp.int32, (BN, 1), 0)
        contrib = jnp.where(row < n_real, cvec * h1, 0.0)
        part = jnp.sum(contrib, axis=0, keepdims=True)

        @pl.when(i == 0)
        def _():
            acc[...] = part

        @pl.when(i > 0)
        def _():
            acc[...] = acc[...] + part

        @pl.when(i == ng - 1)
        def _():
            out_ref[...] = (
                jnp.dot(acc[...], w2_ref[...],
                        preferred_element_type=jnp.float32) * (1.0 / n_real)
                + b2_ref[...]
            )

    return pl.pallas_call(
        body,
        grid=(ng,),
        in_specs=[
            pl.BlockSpec((NC, BN, d_hid), lambda i: (0, i, 0)),
            pl.BlockSpec((BN, d_hid), lambda i: (i, 0)),
            pl.BlockSpec((BN, 1), lambda i: (i, 0)),
            pl.BlockSpec((NC, BN, 1), lambda i: (0, i, 0)),
            pl.BlockSpec((1, d_hid), lambda i: (0, 0)),
            pl.BlockSpec((d_hid, d_out), lambda i: (0, 0)),
            pl.BlockSpec((1, d_out), lambda i: (0, 0)),
        ],
        out_specs=pl.BlockSpec((1, d_out), lambda i: (0, 0)),
        out_shape=jax.ShapeDtypeStruct((1, d_out), jnp.float32),
        scratch_shapes=[pltpu.VMEM((1, d_hid), jnp.float32)],
    )(a2, y, dinv, s3, b1, w2, b2)


def kernel(x, edge_index, batch, W1, b1, W2, b2):
    n, d_in = x.shape
    e = edge_index.shape[1]
    d_hid = W1.shape[1]
    d_out = W2.shape[1]

    n_pad = -(-n // BN) * BN
    ch = -(-e // (NW * K * 6)) * 6
    e_pad = NW * K * ch
    if e_pad > e and n_pad == n:
        n_pad += BN

    if e_pad > e:
        pad_idx = n + (jnp.arange(e_pad - e, dtype=jnp.int32) % (n_pad - n))
        src = jnp.concatenate([edge_index[0], pad_idx])
        dst = jnp.concatenate([edge_index[1], pad_idx])
    else:
        src, dst = edge_index[0], edge_index[1]

    zeros_small = jnp.zeros((n_pad,), jnp.float32)
    zeros_big = jnp.zeros((n_pad, d_hid), jnp.float32)

    src4 = src.reshape(NW, 2, ch // 2, K)
    dst4 = dst.reshape(NW, 2, ch // 2, K)

    deg2 = _make_deg_kernel(n_pad, ch)(dst4, zeros_small)

    x_pad = jnp.pad(x, ((0, n_pad - n), (0, 0)))
    y, dinv = _tc1(x_pad, W1, deg2.reshape(NC, n_pad, 1))

    a2, s2 = _make_edge_kernel(n_pad, ch, d_hid)(
        src4, dst4, y, dinv.reshape(n_pad), zeros_big, zeros_small
    )

    return _tc2(
        a2, y, dinv, s2.reshape(NC, n_pad, 1),
        b1.reshape(1, d_hid), W2, b2.reshape(1, d_out), n
    )


# speedup vs baseline: 47.5910x; 1.0412x over previous
import functools

import jax
import jax.numpy as jnp
from jax import lax
from jax.experimental import pallas as pl
from jax.experimental.pallas import tpu as pltpu
from jax.experimental.pallas import tpu_sc as plsc

NC = 2
NS = 16
NW = NC * NS
K = 80
BN = 1024
LN = 128


def _sc_mesh():
    return plsc.VectorSubcoreMesh(
        core_axis_name="c", subcore_axis_name="s", num_cores=NC, num_subcores=NS
    )


def _make_deg_kernel(n_pad, ch):
    rows = n_pad // NS
    ch2 = ch // 2

    @functools.partial(
        pl.kernel,
        out_type=jax.ShapeDtypeStruct((NC, n_pad), jnp.float32),
        mesh=_sc_mesh(),
        scratch_types=[
            pltpu.VMEM((2, ch2, K), jnp.int32),
            pltpu.VMEM((K,), jnp.float32),
            pltpu.VMEM_SHARED((n_pad,), jnp.float32),
            pltpu.SemaphoreType.DMA,
        ],
    )
    def deg_kernel(dst_hbm, zeros_hbm, deg_out, idx_v, ones_v, acc_sh, sem):
        c = lax.axis_index("c")
        s = lax.axis_index("s")
        w = c * NS + s
        pltpu.sync_copy(zeros_hbm.at[pl.ds(s * rows, rows)],
                        acc_sh.at[pl.ds(s * rows, rows)])
        for i in range(K // 16):
            ones_v[pl.ds(i * 16, 16)] = jnp.ones((16,), jnp.float32)
        pltpu.sync_copy(dst_hbm.at[w], idx_v)
        plsc.subcore_barrier()

        def body(j, carry):
            pltpu.async_copy(ones_v, acc_sh.at[idx_v.at[j // ch2, j % ch2]],
                             sem, add=True)
            return carry

        lax.fori_loop(0, ch, body, 0)

        def drain(j, carry):
            pltpu.make_async_copy(ones_v,
                                  acc_sh.at[idx_v.at[j // ch2, j % ch2]],
                                  sem).wait()
            return carry

        lax.fori_loop(0, ch, drain, 0)
        plsc.subcore_barrier()
        pltpu.sync_copy(acc_sh.at[pl.ds(s * rows, rows)],
                        deg_out.at[c, pl.ds(s * rows, rows)])

    return deg_kernel


def _make_edge_kernel(n_pad, ch, d_hid):
    rows = n_pad // NS
    ch2 = ch // 2
    nt = ch2 // 3

    @functools.partial(
        pl.kernel,
        out_type=(
            jax.ShapeDtypeStruct((NC, n_pad, d_hid), jnp.float32),
            jax.ShapeDtypeStruct((NC, n_pad), jnp.float32),
        ),
        mesh=_sc_mesh(),
        scratch_types=[
            pltpu.VMEM((ch2, K), jnp.int32),
            pltpu.VMEM((ch2, K), jnp.int32),
            pltpu.VMEM((3, K, d_hid), jnp.float32),
            pltpu.VMEM((3, K), jnp.float32),
            pltpu.VMEM_SHARED((n_pad, d_hid), jnp.float32),
            pltpu.VMEM_SHARED((n_pad,), jnp.float32),
            pltpu.SemaphoreType.DMA,
            pltpu.SemaphoreType.DMA,
            pltpu.SemaphoreType.DMA,
            pltpu.SemaphoreType.DMA,
            pltpu.SemaphoreType.DMA,
            pltpu.SemaphoreType.DMA,
        ],
    )
    def edge_kernel(src_hbm, dst_hbm, y_hbm, dinv_hbm, zbig_hbm, zsmall_hbm,
                    a_out, s_out, srcv, dstv, rows_v, dval_v, a_sh, s_sh,
                    gsem0, gsem1, gsem2, ssem0, ssem1, ssem2):
        c = lax.axis_index("c")
        s = lax.axis_index("s")
        w = c * NS + s
        pltpu.sync_copy(zbig_hbm.at[pl.ds(s * rows, rows)],
                        a_sh.at[pl.ds(s * rows, rows)])
        pltpu.sync_copy(zsmall_hbm.at[pl.ds(s * rows, rows)],
                        s_sh.at[pl.ds(s * rows, rows)])
        plsc.subcore_barrier()

        gsem = (gsem0, gsem1, gsem2)
        ssem = (ssem0, ssem1, ssem2)

        def issue_gather(j, b):
            pltpu.async_copy(y_hbm.at[srcv.at[j]], rows_v.at[b], gsem[b])
            pltpu.async_copy(dinv_hbm.at[dstv.at[j]], dval_v.at[b], gsem[b])

        def wait_gather(j, b):
            pltpu.make_async_copy(y_hbm.at[srcv.at[j]], rows_v.at[b],
                                  gsem[b]).wait()
            pltpu.make_async_copy(dinv_hbm.at[dstv.at[j]], dval_v.at[b],
                                  gsem[b]).wait()

        def issue_scatter(j, b):
            pltpu.async_copy(rows_v.at[b], a_sh.at[dstv.at[j]], ssem[b],
                             add=True)
            pltpu.async_copy(dval_v.at[b], s_sh.at[srcv.at[j]], ssem[b],
                             add=True)

        def wait_scatter(j, b):
            pltpu.make_async_copy(rows_v.at[b], a_sh.at[dstv.at[j]],
                                  ssem[b]).wait()
            pltpu.make_async_copy(dval_v.at[b], s_sh.at[srcv.at[j]],
                                  ssem[b]).wait()

        for p in range(2):
            pltpu.sync_copy(src_hbm.at[w, p], srcv)
            pltpu.sync_copy(dst_hbm.at[w, p], dstv)
            issue_gather(0, 0)
            issue_gather(1, 1)

            def body(t, carry):
                j0 = 3 * t
                wait_gather(j0, 0)
                issue_scatter(j0, 0)

                @pl.when(t > 0)
                def _():
                    wait_scatter(j0 - 1, 2)

                issue_gather(j0 + 2, 2)
                wait_gather(j0 + 1, 1)
                issue_scatter(j0 + 1, 1)
                wait_scatter(j0, 0)

                @pl.when(t < nt - 1)
                def _():
                    issue_gather(j0 + 3, 0)

                wait_gather(j0 + 2, 2)
                issue_scatter(j0 + 2, 2)
                wait_scatter(j0 + 1, 1)

                @pl.when(t < nt - 1)
                def _():
                    issue_gather(j0 + 4, 1)

                return carry

            lax.fori_loop(0, nt, body, 0)
            wait_scatter(ch2 - 1, 2)

        plsc.subcore_barrier()
        pltpu.sync_copy(a_sh.at[pl.ds(s * rows, rows)],
                        a_out.at[c, pl.ds(s * rows, rows)])
        pltpu.sync_copy(s_sh.at[pl.ds(s * rows, rows)],
                        s_out.at[c, pl.ds(s * rows, rows)])

    return edge_kernel


def _tc1(x_pad, w1, deg3):
    n_pad, d_in = x_pad.shape
    d_hid = w1.shape[1]

    def body(x_ref, w1_ref, deg_ref, y_ref, dinv_ref):
        dinv = lax.rsqrt(deg_ref[0] + deg_ref[1] + 1.0)
        xw = jnp.dot(x_ref[...], w1_ref[...], preferred_element_type=jnp.float32)
        y_ref[...] = dinv * xw
        dinv_ref[...] = dinv

    return pl.pallas_call(
        body,
        grid=(n_pad // BN,),
        in_specs=[
            pl.BlockSpec((BN, d_in), lambda i: (i, 0)),
            pl.BlockSpec((d_in, d_hid), lambda i: (0, 0)),
            pl.BlockSpec((NC, BN, 1), lambda i: (0, i, 0)),
        ],
        out_specs=[
            pl.BlockSpec((BN, d_hid), lambda i: (i, 0)),
            pl.BlockSpec((BN, 1), lambda i: (i, 0)),
        ],
        out_shape=[
            jax.ShapeDtypeStruct((n_pad, d_hid), jnp.float32),
            jax.ShapeDtypeStruct((n_pad, 1), jnp.float32),
        ],
    )(x_pad, w1, deg3)


def _tc2(a2, y, dinv, s3, b1, w2, b2, n_real):
    nc, n_pad, d_hid = a2.shape
    d_out = w2.shape[1]
    ng = n_pad // BN

    def body(a_ref, y_ref, dinv_ref, s_ref, b1_ref, w2_ref, b2_ref,
             out_ref, acc):
        i = pl.program_id(0)
        a = a_ref[0] + a_ref[1]
        dv = dinv_ref[...]
        h1 = jnp.maximum(dv * (a + y_ref[...]) + b1_ref[...], 0.0)
        sv = s_ref[0] + s_ref[1]
        cvec = dv * (sv + dv)
        row = i * BN + lax.broadcasted_iota(jnp.int32, (BN, 1), 0)
        contrib = jnp.where(row < n_real, cvec * h1, 0.0)
        part = jnp.sum(contrib, axis=0, keepdims=True)

        @pl.when(i == 0)
        def _():
            acc[...] = part

        @pl.when(i > 0)
        def _():
            acc[...] = acc[...] + part

        @pl.when(i == ng - 1)
        def _():
            out_ref[...] = (
                jnp.dot(acc[...], w2_ref[...],
                        preferred_element_type=jnp.float32) * (1.0 / n_real)
                + b2_ref[...]
            )

    return pl.pallas_call(
        body,
        grid=(ng,),
        in_specs=[
            pl.BlockSpec((NC, BN, d_hid), lambda i: (0, i, 0)),
            pl.BlockSpec((BN, d_hid), lambda i: (i, 0)),
            pl.BlockSpec((BN, 1), lambda i: (i, 0)),
            pl.BlockSpec((NC, BN, 1), lambda i: (0, i, 0)),
            pl.BlockSpec((1, d_hid), lambda i: (0, 0)),
            pl.BlockSpec((d_hid, d_out), lambda i: (0, 0)),
            pl.BlockSpec((1, d_out), lambda i: (0, 0)),
        ],
        out_specs=pl.BlockSpec((1, d_out), lambda i: (0, 0)),
        out_shape=jax.ShapeDtypeStruct((1, d_out), jnp.float32),
        scratch_shapes=[pltpu.VMEM((1, d_hid), jnp.float32)],
    )(a2, y, dinv, s3, b1, w2, b2)


def kernel(x, edge_index, batch, W1, b1, W2, b2):
    n, d_in = x.shape
    e = edge_index.shape[1]
    d_hid = W1.shape[1]
    d_out = W2.shape[1]

    n_pad = -(-n // BN) * BN
    ch = -(-e // (NW * K * 6)) * 6
    e_pad = NW * K * ch
    if e_pad > e and n_pad == n:
        n_pad += BN

    if e_pad > e:
        pad_idx = n + (jnp.arange(e_pad - e, dtype=jnp.int32) % (n_pad - n))
        src = jnp.concatenate([edge_index[0], pad_idx])
        dst = jnp.concatenate([edge_index[1], pad_idx])
    else:
        src, dst = edge_index[0], edge_index[1]

    zeros_small = jnp.zeros((n_pad,), jnp.float32)
    zeros_big = jnp.zeros((n_pad, d_hid), jnp.float32)

    src4 = src.reshape(NW, 2, ch // 2, K)
    dst4 = dst.reshape(NW, 2, ch // 2, K)

    deg2 = _make_deg_kernel(n_pad, ch)(dst4, zeros_small)

    x_pad = jnp.pad(x, ((0, n_pad - n), (0, 0)))
    y, dinv = _tc1(x_pad, W1, deg2.reshape(NC, n_pad, 1))

    a2, s2 = _make_edge_kernel(n_pad, ch, d_hid)(
        src4, dst4, y, dinv.reshape(n_pad), zeros_big, zeros_small
    )

    return _tc2(
        a2, y, dinv, s2.reshape(NC, n_pad, 1),
        b1.reshape(1, d_hid), W2, b2.reshape(1, d_out), n
    )


# speedup vs baseline: 49.6636x; 1.0435x over previous
import functools

import jax
import jax.numpy as jnp
import numpy as np
from jax import lax
from jax.experimental import pallas as pl
from jax.experimental.pallas import tpu as pltpu
from jax.experimental.pallas import tpu_sc as plsc

NC = 2
NS = 16
NW = NC * NS
K = 80
BN = 1024
LN = 128


def _sc_mesh():
    return plsc.VectorSubcoreMesh(
        core_axis_name="c", subcore_axis_name="s", num_cores=NC, num_subcores=NS
    )


def _make_deg_kernel(n_pad, ch):
    rows = n_pad // NS
    ch2 = ch // 2

    @functools.partial(
        pl.kernel,
        out_type=jax.ShapeDtypeStruct((NC, n_pad), jnp.float32),
        mesh=_sc_mesh(),
        scratch_types=[
            pltpu.VMEM((2, ch2, K), jnp.int32),
            pltpu.VMEM((K,), jnp.float32),
            pltpu.VMEM_SHARED((n_pad,), jnp.float32),
            pltpu.SemaphoreType.DMA,
        ],
    )
    def deg_kernel(dst_hbm, zeros_hbm, deg_out, idx_v, ones_v, acc_sh, sem):
        c = lax.axis_index("c")
        s = lax.axis_index("s")
        w = c * NS + s
        pltpu.sync_copy(zeros_hbm.at[pl.ds(s * rows, rows)],
                        acc_sh.at[pl.ds(s * rows, rows)])
        for i in range(K // 16):
            ones_v[pl.ds(i * 16, 16)] = jnp.ones((16,), jnp.float32)
        pltpu.sync_copy(dst_hbm.at[w], idx_v)
        plsc.subcore_barrier()

        def body(j, carry):
            pltpu.async_copy(ones_v, acc_sh.at[idx_v.at[j // ch2, j % ch2]],
                             sem, add=True)
            return carry

        lax.fori_loop(0, ch, body, 0)

        def drain(j, carry):
            pltpu.make_async_copy(ones_v,
                                  acc_sh.at[idx_v.at[j // ch2, j % ch2]],
                                  sem).wait()
            return carry

        lax.fori_loop(0, ch, drain, 0)
        plsc.subcore_barrier()
        pltpu.sync_copy(acc_sh.at[pl.ds(s * rows, rows)],
                        deg_out.at[c, pl.ds(s * rows, rows)])

    return deg_kernel


def _make_edge_kernel(n_pad, ch, d_hid):
    rows = n_pad // NS
    ch2 = ch // 2
    nt = ch2 // 3

    @functools.partial(
        pl.kernel,
        out_type=(
            jax.ShapeDtypeStruct((NC, n_pad, d_hid), jnp.float32),
            jax.ShapeDtypeStruct((NC, n_pad), jnp.float32),
        ),
        mesh=_sc_mesh(),
        scratch_types=[
            pltpu.VMEM((ch2, K), jnp.int32),
            pltpu.VMEM((ch2, K), jnp.int32),
            pltpu.VMEM((3, K, d_hid), jnp.float32),
            pltpu.VMEM((3, K), jnp.float32),
            pltpu.VMEM_SHARED((n_pad, d_hid), jnp.float32),
            pltpu.VMEM_SHARED((n_pad,), jnp.float32),
            pltpu.SemaphoreType.DMA,
            pltpu.SemaphoreType.DMA,
            pltpu.SemaphoreType.DMA,
            pltpu.SemaphoreType.DMA,
            pltpu.SemaphoreType.DMA,
            pltpu.SemaphoreType.DMA,
        ],
    )
    def edge_kernel(src_hbm, dst_hbm, y_hbm, dinv_hbm, zbig_hbm, zsmall_hbm,
                    a_out, s_out, srcv, dstv, rows_v, dval_v, a_sh, s_sh,
                    gsem0, gsem1, gsem2, ssem0, ssem1, ssem2):
        c = lax.axis_index("c")
        s = lax.axis_index("s")
        w = c * NS + s
        pltpu.sync_copy(zbig_hbm.at[pl.ds(s * rows, rows)],
                        a_sh.at[pl.ds(s * rows, rows)])
        pltpu.sync_copy(zsmall_hbm.at[pl.ds(s * rows, rows)],
                        s_sh.at[pl.ds(s * rows, rows)])
        plsc.subcore_barrier()

        gsem = (gsem0, gsem1, gsem2)
        ssem = (ssem0, ssem1, ssem2)

        def issue_gather(j, b):
            pltpu.async_copy(y_hbm.at[srcv.at[j]], rows_v.at[b], gsem[b])
            pltpu.async_copy(dinv_hbm.at[dstv.at[j]], dval_v.at[b], gsem[b])

        def wait_gather(j, b):
            pltpu.make_async_copy(y_hbm.at[srcv.at[j]], rows_v.at[b],
                                  gsem[b]).wait()
            pltpu.make_async_copy(dinv_hbm.at[dstv.at[j]], dval_v.at[b],
                                  gsem[b]).wait()

        def issue_scatter(j, b):
            pltpu.async_copy(rows_v.at[b], a_sh.at[dstv.at[j]], ssem[b],
                             add=True)
            pltpu.async_copy(dval_v.at[b], s_sh.at[srcv.at[j]], ssem[b],
                             add=True)

        def wait_scatter(j, b):
            pltpu.make_async_copy(rows_v.at[b], a_sh.at[dstv.at[j]],
                                  ssem[b]).wait()
            pltpu.make_async_copy(dval_v.at[b], s_sh.at[srcv.at[j]],
                                  ssem[b]).wait()

        for p in range(2):
            pltpu.sync_copy(src_hbm.at[w, p], srcv)
            pltpu.sync_copy(dst_hbm.at[w, p], dstv)
            issue_gather(0, 0)
            issue_gather(1, 1)

            def body(t, carry):
                j0 = 3 * t
                wait_gather(j0, 0)
                issue_scatter(j0, 0)

                @pl.when(t > 0)
                def _():
                    wait_scatter(j0 - 1, 2)

                issue_gather(j0 + 2, 2)
                wait_gather(j0 + 1, 1)
                issue_scatter(j0 + 1, 1)
                wait_scatter(j0, 0)

                @pl.when(t < nt - 1)
                def _():
                    issue_gather(j0 + 3, 0)

                wait_gather(j0 + 2, 2)
                issue_scatter(j0 + 2, 2)
                wait_scatter(j0 + 1, 1)

                @pl.when(t < nt - 1)
                def _():
                    issue_gather(j0 + 4, 1)

                return carry

            lax.fori_loop(0, nt, body, 0)
            wait_scatter(ch2 - 1, 2)

        plsc.subcore_barrier()
        pltpu.sync_copy(a_sh.at[pl.ds(s * rows, rows)],
                        a_out.at[c, pl.ds(s * rows, rows)])
        pltpu.sync_copy(s_sh.at[pl.ds(s * rows, rows)],
                        s_out.at[c, pl.ds(s * rows, rows)])

    return edge_kernel


def _tc1(x_pad, w1, deg3):
    n_pad, d_in = x_pad.shape
    d_hid = w1.shape[1]

    def body(x_ref, w1_ref, deg_ref, y_ref, dinv_ref):
        dinv = lax.rsqrt(deg_ref[0] + deg_ref[1] + 1.0)
        xw = jnp.dot(x_ref[...], w1_ref[...], preferred_element_type=jnp.float32)
        y_ref[...] = dinv * xw
        dinv_ref[...] = dinv

    return pl.pallas_call(
        body,
        grid=(n_pad // BN,),
        in_specs=[
            pl.BlockSpec((BN, d_in), lambda i: (i, 0)),
            pl.BlockSpec((d_in, d_hid), lambda i: (0, 0)),
            pl.BlockSpec((NC, BN, 1), lambda i: (0, i, 0)),
        ],
        out_specs=[
            pl.BlockSpec((BN, d_hid), lambda i: (i, 0)),
            pl.BlockSpec((BN, 1), lambda i: (i, 0)),
        ],
        out_shape=[
            jax.ShapeDtypeStruct((n_pad, d_hid), jnp.float32),
            jax.ShapeDtypeStruct((n_pad, 1), jnp.float32),
        ],
    )(x_pad, w1, deg3)


def _tc2(a2, y, dinv, s3, b1, w2, b2, n_real):
    nc, n_pad, d_hid = a2.shape
    d_out = w2.shape[1]
    ng = n_pad // BN

    def body(a_ref, y_ref, dinv_ref, s_ref, b1_ref, w2_ref, b2_ref,
             out_ref, acc):
        i = pl.program_id(0)
        a = a_ref[0] + a_ref[1]
        dv = dinv_ref[...]
        h1 = jnp.maximum(dv * (a + y_ref[...]) + b1_ref[...], 0.0)
        sv = s_ref[0] + s_ref[1]
        cvec = dv * (sv + dv)
        row = i * BN + lax.broadcasted_iota(jnp.int32, (BN, 1), 0)
        contrib = jnp.where(row < n_real, cvec * h1, 0.0)
        part = jnp.sum(contrib, axis=0, keepdims=True)

        @pl.when(i == 0)
        def _():
            acc[...] = part

        @pl.when(i > 0)
        def _():
            acc[...] = acc[...] + part

        @pl.when(i == ng - 1)
        def _():
            out_ref[...] = (
                jnp.dot(acc[...], w2_ref[...],
                        preferred_element_type=jnp.float32) * (1.0 / n_real)
                + b2_ref[...]
            )

    return pl.pallas_call(
        body,
        grid=(ng,),
        in_specs=[
            pl.BlockSpec((NC, BN, d_hid), lambda i: (0, i, 0)),
            pl.BlockSpec((BN, d_hid), lambda i: (i, 0)),
            pl.BlockSpec((BN, 1), lambda i: (i, 0)),
            pl.BlockSpec((NC, BN, 1), lambda i: (0, i, 0)),
            pl.BlockSpec((1, d_hid), lambda i: (0, 0)),
            pl.BlockSpec((d_hid, d_out), lambda i: (0, 0)),
            pl.BlockSpec((1, d_out), lambda i: (0, 0)),
        ],
        out_specs=pl.BlockSpec((1, d_out), lambda i: (0, 0)),
        out_shape=jax.ShapeDtypeStruct((1, d_out), jnp.float32),
        scratch_shapes=[pltpu.VMEM((1, d_hid), jnp.float32)],
    )(a2, y, dinv, s3, b1, w2, b2)


def kernel(x, edge_index, batch, W1, b1, W2, b2):
    n, d_in = x.shape
    e = edge_index.shape[1]
    d_hid = W1.shape[1]
    d_out = W2.shape[1]

    n_pad = -(-n // BN) * BN
    ch = -(-e // (NW * K * 6)) * 6
    e_pad = NW * K * ch
    if e_pad > e and n_pad == n:
        n_pad += BN

    if e_pad > e:
        pad_idx = jnp.asarray(
            n + (np.arange(e_pad - e, dtype=np.int32) % (n_pad - n)),
            jnp.int32)
        src = jnp.concatenate([edge_index[0], pad_idx])
        dst = jnp.concatenate([edge_index[1], pad_idx])
    else:
        src, dst = edge_index[0], edge_index[1]

    zeros_small = jnp.zeros((n_pad,), jnp.float32)
    zeros_big = jnp.zeros((n_pad, d_hid), jnp.float32)

    src4 = src.reshape(NW, 2, ch // 2, K)
    dst4 = dst.reshape(NW, 2, ch // 2, K)

    deg2 = _make_deg_kernel(n_pad, ch)(dst4, zeros_small)

    x_pad = jnp.pad(x, ((0, n_pad - n), (0, 0)))
    y, dinv = _tc1(x_pad, W1, deg2.reshape(NC, n_pad, 1))

    a2, s2 = _make_edge_kernel(n_pad, ch, d_hid)(
        src4, dst4, y, dinv.reshape(n_pad), zeros_big, zeros_small
    )

    return _tc2(
        a2, y, dinv, s2.reshape(NC, n_pad, 1),
        b1.reshape(1, d_hid), W2, b2.reshape(1, d_out), n
    )
